# baseline (speedup 1.0000x reference)
import functools

import jax
import jax.numpy as jnp
from jax import lax
from jax.experimental import pallas as pl
from jax.experimental.pallas import tpu as pltpu
from jax.experimental.pallas import tpu_sc as plsc

B = 16384
D = 64
HIDDEN = 256
NC = 2
NS = 16
NW = NC * NS
NTILE = 7813
TPW = 244
W = 24
NB = 11
CAP = 768
LANES = 16


def _scan_last(vec):
    return plsc.cumsum(vec)[LANES - 1]


def _filter_pass(src_hbm, stage, wl_j, wl_i, t0, t1):
    cnt = 0
    for half in range(2):
        pltpu.sync_copy(src_hbm.at[pl.ds(half * (B // 2), B // 2)], stage)

        def chunk(c, cnt):
            j = stage[pl.ds(pl.multiple_of(c * LANES, 8), LANES)]
            t = j >> 7
            mask = (t >= t0) & (t < t1)
            mi = mask.astype(jnp.int32)
            rank = plsc.cumsum(mi) - mi
            pos = rank + cnt
            mask = mask & (pos < CAP)
            ivec = lax.iota(jnp.int32, LANES) + (half * (B // 2) + c * LANES)
            plsc.store_scatter(wl_j, [pos], j, mask=mask)
            plsc.store_scatter(wl_i, [pos], ivec, mask=mask)
            return cnt + _scan_last(mi)

        cnt = lax.fori_loop(0, (B // 2) // LANES, chunk, cnt)
    return cnt


def _bucket_sort(wl_j, wl_i, wl2_j, wl2_i, counts, cursor, cnt, t0):
    zeros = jnp.zeros((LANES,), jnp.int32)
    counts[...] = zeros
    pad = jnp.full((LANES,), B, jnp.int32)
    for c in range(CAP // LANES):
        wl2_i[pl.ds(c * LANES, LANES)] = pad
    ones = jnp.full((LANES,), 1, jnp.int32)

    def count_chunk(c, carry):
        off = pl.multiple_of(c * LANES, 8)
        j = wl_j[pl.ds(off, LANES)]
        valid = (lax.iota(jnp.int32, LANES) + c * LANES) < cnt
        qv = ((j >> 7) - t0) // W
        plsc.addupdate_scatter(counts, [qv], ones, mask=valid)
        return carry

    lax.fori_loop(0, CAP // LANES, count_chunk, 0)
    cvec = counts[...]
    incl = plsc.cumsum(cvec)
    boff = incl - cvec
    cursor[...] = boff

    def place_chunk(c, carry):
        off = pl.multiple_of(c * LANES, 8)
        j = wl_j[pl.ds(off, LANES)]
        i = wl_i[pl.ds(off, LANES)]
        valid = (lax.iota(jnp.int32, LANES) + c * LANES) < cnt
        qv = ((j >> 7) - t0) // W
        base = plsc.load_gather(cursor, [qv], mask=valid)
        rank = zeros
        for q in range(NB):
            mq = (valid & (qv == q)).astype(jnp.int32)
            rank = rank + (plsc.cumsum(mq) - mq) * mq
        pos = base + rank
        plsc.store_scatter(wl2_j, [pos], j, mask=valid)
        plsc.store_scatter(wl2_i, [pos], i, mask=valid)
        plsc.addupdate_scatter(cursor, [qv], ones, mask=valid)
        return carry

    lax.fori_loop(0, CAP // LANES, place_chunk, 0)
    return boff


def _extract_window(slot3, par, wl2_j, wstart, e_lo, e_hi, outbuf, dslab):
    iot = lax.iota(jnp.int32, LANES)
    hi_half = (iot >= 8).astype(jnp.int32)
    lo_half = iot < 8
    pvec = jnp.full((LANES,), par, jnp.int32)

    def pair(p, carry):
        e = e_lo + 2 * p
        mask = lo_half | jnp.broadcast_to((e + 1) < e_hi, (LANES,))
        evec = e + hi_half
        jv = plsc.load_gather(wl2_j, [evec], mask=mask)
        tl = (jv >> 7) - wstart
        r = jv & 127
        jj = tl * 128 + r
        vals = plsc.load_gather(slot3, [pvec, iot & 7, jj], mask=mask)
        opos = evec * D + dslab * 8 + (iot & 7)
        plsc.store_scatter(outbuf, [opos], vals, mask=mask)
        return carry

    npairs = (e_hi - e_lo + 1) // 2
    lax.fori_loop(0, npairs, pair, 0)


def _stream_table(src, out_rows, wl_j, wl_i, wl2_j, wl2_i, stage, counts,
                  cursor, slot3, outbuf, sems, t0, t1, idx_hbm, wid):
    cnt = _filter_pass(idx_hbm, stage, wl_j, wl_i, t0, t1)
    boff = _bucket_sort(wl_j, wl_i, wl2_j, wl2_i, counts, cursor, cnt, t0)
    bvec = boff
    cvec = counts[...]

    for b in range(NB):
        e_lo = bvec[b]
        e_hi = e_lo + cvec[b]
        wstart = jnp.minimum(t0 + b * W, NTILE - W)
        col = pl.multiple_of(wstart * 128, 128)

        def fire(dslab, par):
            d8 = pl.multiple_of(dslab * 8, 8)
            pltpu.async_copy(src.at[pl.ds(d8, 8), pl.ds(col, W * 128)],
                             slot3.at[par], sems[par])

        def drain(par):
            pltpu.make_async_copy(src.at[pl.ds(0, 8), pl.ds(0, W * 128)],
                                  slot3.at[par], sems[par]).wait()

        @pl.when(e_hi > e_lo)
        def _():
            fire(0, 0)

            def mbody(m, carry):
                fire(2 * m + 1, 1)
                drain(0)
                _extract_window(slot3, 0, wl2_j, wstart, e_lo, e_hi, outbuf, 2 * m)

                @pl.when(m < 3)
                def _():
                    fire(2 * m + 2, 0)
                drain(1)
                _extract_window(slot3, 1, wl2_j, wstart, e_lo, e_hi, outbuf, 2 * m + 1)
                return carry

            lax.fori_loop(0, 4, mbody, 0)

    pltpu.sync_copy(outbuf, out_rows.at[pl.ds(wid * (CAP * D), CAP * D)])


def _gather_body(u_hbm, v_hbm, ut_hbm, vt_hbm,
                 rows_u, rows_v, wli_u, wli_v,
                 stage, wl_j, wl_i, wl2_j, wl2_i, counts, cursor,
                 slot3, outbuf, sem_a, sem_b):
    wid = lax.axis_index("s") * NC + lax.axis_index("c")
    t0 = wid * TPW + jnp.minimum(wid, 5)
    t1 = t0 + TPW + jnp.where(wid < 5, 1, 0)
    sems = (sem_a, sem_b)
    _stream_table(ut_hbm, rows_u, wl_j, wl_i, wl2_j, wl2_i, stage, counts,
                  cursor, slot3, outbuf, sems, t0, t1, u_hbm, wid)
    pltpu.sync_copy(wl2_i, wli_u.at[pl.ds(wid * CAP, CAP)])
    _stream_table(vt_hbm, rows_v, wl_j, wl_i, wl2_j, wl2_i, stage, counts,
                  cursor, slot3, outbuf, sems, t0, t1, v_hbm, wid)
    pltpu.sync_copy(wl2_i, wli_v.at[pl.ds(wid * CAP, CAP)])


def _make_gather():
    mesh = plsc.VectorSubcoreMesh(core_axis_name="c", subcore_axis_name="s")
    return functools.partial(
        pl.kernel, mesh=mesh,
        compiler_params=pltpu.CompilerParams(use_tc_tiling_on_sc=True,
                                             needs_layout_passes=False),
        out_type=[jax.ShapeDtypeStruct((NW * CAP * D,), jnp.float32),
                  jax.ShapeDtypeStruct((NW * CAP * D,), jnp.float32),
                  jax.ShapeDtypeStruct((NW * CAP,), jnp.int32),
                  jax.ShapeDtypeStruct((NW * CAP,), jnp.int32)],
        scratch_types=[
            pltpu.VMEM((B // 2,), jnp.int32),
            pltpu.VMEM((CAP,), jnp.int32),
            pltpu.VMEM((CAP,), jnp.int32),
            pltpu.VMEM((CAP,), jnp.int32),
            pltpu.VMEM((CAP,), jnp.int32),
            pltpu.VMEM((LANES,), jnp.int32),
            pltpu.VMEM((LANES,), jnp.int32),
            pltpu.VMEM((2, 8, W * 128), jnp.float32),
            pltpu.VMEM((CAP * D,), jnp.float32),
            pltpu.SemaphoreType.DMA,
            pltpu.SemaphoreType.DMA,
        ],
    )(_gather_body)


_sc_gather = _make_gather()


def _scatter_body(rows_u, rows_v, wli_u, wli_v, ug_hbm, vg_hbm,
                  rowbuf, idxbuf, sem):
    wid = lax.axis_index("s") * NC + lax.axis_index("c")
    for rows, wli, out in ((rows_u, wli_u, ug_hbm), (rows_v, wli_v, vg_hbm)):
        pltpu.sync_copy(rows.at[pl.ds(wid * CAP, CAP)], rowbuf)
        pltpu.sync_copy(wli.at[wid], idxbuf)
        copies = []
        for k in range(CAP // 128):
            copies.append(pltpu.async_copy(
                rowbuf.at[pl.ds(k * 128, 128)], out.at[idxbuf.at[k]], sem))
        for c in copies:
            c.wait()


def _make_scatter():
    mesh = plsc.VectorSubcoreMesh(core_axis_name="c", subcore_axis_name="s")
    return functools.partial(
        pl.kernel, mesh=mesh,
        compiler_params=pltpu.CompilerParams(use_tc_tiling_on_sc=False),
        out_type=[jax.ShapeDtypeStruct((B + 128, D), jnp.float32),
                  jax.ShapeDtypeStruct((B + 128, D), jnp.float32)],
        scratch_types=[
            pltpu.VMEM((CAP, D), jnp.float32),
            pltpu.VMEM((CAP // 128, 128), jnp.int32),
            pltpu.SemaphoreType.DMA,
        ],
    )(_scatter_body)


_sc_scatter = _make_scatter()


def _mlp_body(xu_ref, xv_ref, w1a_ref, w1b_ref, b1_ref, w2_ref, b2_ref, o_ref):
    xu = jnp.maximum(xu_ref[...], 0.0)
    xv = jnp.maximum(xv_ref[...], 0.0)
    h = jnp.dot(xu, w1a_ref[...], preferred_element_type=jnp.float32)
    h = h + jnp.dot(xv, w1b_ref[...], preferred_element_type=jnp.float32)
    h = jnp.maximum(h + b1_ref[...], 0.0)
    o_ref[...] = jnp.dot(h, w2_ref[...], preferred_element_type=jnp.float32) + b2_ref[...]


BB = 2048


def _mlp(ug, vg, w1a, w1b, b1, w2, b2):
    grid = (B // BB,)
    return pl.pallas_call(
        _mlp_body,
        grid=grid,
        in_specs=[
            pl.BlockSpec((BB, D), lambda i: (i, 0)),
            pl.BlockSpec((BB, D), lambda i: (i, 0)),
            pl.BlockSpec((D, HIDDEN), lambda i: (0, 0)),
            pl.BlockSpec((D, HIDDEN), lambda i: (0, 0)),
            pl.BlockSpec((1, HIDDEN), lambda i: (0, 0)),
            pl.BlockSpec((HIDDEN, 1), lambda i: (0, 0)),
            pl.BlockSpec((1, 1), lambda i: (0, 0)),
        ],
        out_specs=pl.BlockSpec((BB, 1), lambda i: (i, 0)),
        out_shape=jax.ShapeDtypeStruct((B, 1), jnp.float32),
    )(ug, vg, w1a, w1b, b1, w2, b2)


def kernel(u, v, user_emb, like_emb, W1, b1, W2, b2):
    rows_u, rows_v, wli_u, wli_v = _sc_gather(u, v, user_emb.T, like_emb.T)
    ug, vg = _sc_scatter(rows_u.reshape(NW * CAP, D), rows_v.reshape(NW * CAP, D),
                         wli_u.reshape(NW, CAP // 128, 128),
                         wli_v.reshape(NW, CAP // 128, 128))
    return _mlp(ug, vg, W1[:D], W1[D:], b1.reshape(1, HIDDEN),
                W2, b2.reshape(1, 1))


# speedup vs baseline: 1.6524x; 1.6524x over previous
import functools

import jax
import jax.numpy as jnp
from jax import lax
from jax.experimental import pallas as pl
from jax.experimental.pallas import tpu as pltpu
from jax.experimental.pallas import tpu_sc as plsc

B = 16384
D = 64
HIDDEN = 256
NC = 2
NS = 16
NW = NC * NS
NTILE = 7813
TPW = 244
W = 24
NB = 11
CAP = 768
LANES = 16


def _scan_last(vec):
    return plsc.cumsum(vec)[LANES - 1]


def _filter_pass(src_hbm, stage, wl_j, wl_i, t0, t1):
    cnt = 0
    for half in range(2):
        pltpu.sync_copy(src_hbm.at[pl.ds(half * (B // 2), B // 2)], stage)

        def chunk(c, cnt):
            j = stage[pl.ds(pl.multiple_of(c * LANES, 8), LANES)]
            t = j >> 7
            mask = (t >= t0) & (t < t1)
            mi = mask.astype(jnp.int32)
            rank = plsc.cumsum(mi) - mi
            pos = rank + cnt
            mask = mask & (pos < CAP)
            ivec = lax.iota(jnp.int32, LANES) + (half * (B // 2) + c * LANES)
            plsc.store_scatter(wl_j, [pos], j, mask=mask)
            plsc.store_scatter(wl_i, [pos], ivec, mask=mask)
            return cnt + _scan_last(mi)

        cnt = lax.fori_loop(0, (B // 2) // LANES, chunk, cnt)
    return cnt


def _bucket_sort(wl_j, wl_i, wl2_j, wl2_i, counts, cursor, cnt, t0, wid):
    zeros = jnp.zeros((LANES,), jnp.int32)
    counts[...] = zeros
    pad = jnp.full((LANES,), B, jnp.int32) + wid
    for c in range(CAP // LANES):
        wl2_i[pl.ds(c * LANES, LANES)] = pad
    ones = jnp.full((LANES,), 1, jnp.int32)

    def count_chunk(c, carry):
        off = pl.multiple_of(c * LANES, 8)
        j = wl_j[pl.ds(off, LANES)]
        valid = (lax.iota(jnp.int32, LANES) + c * LANES) < cnt
        qv = ((j >> 7) - t0) // W
        plsc.addupdate_scatter(counts, [qv], ones, mask=valid)
        return carry

    lax.fori_loop(0, CAP // LANES, count_chunk, 0)
    cvec = counts[...]
    incl = plsc.cumsum(cvec)
    boff = incl - cvec
    cursor[...] = boff

    def place_chunk(c, carry):
        off = pl.multiple_of(c * LANES, 8)
        j = wl_j[pl.ds(off, LANES)]
        i = wl_i[pl.ds(off, LANES)]
        valid = (lax.iota(jnp.int32, LANES) + c * LANES) < cnt
        qv = ((j >> 7) - t0) // W
        base = plsc.load_gather(cursor, [qv], mask=valid)
        rank = zeros
        for q in range(NB):
            mq = (valid & (qv == q)).astype(jnp.int32)
            rank = rank + (plsc.cumsum(mq) - mq) * mq
        pos = base + rank
        plsc.store_scatter(wl2_j, [pos], j, mask=valid)
        plsc.store_scatter(wl2_i, [pos], i, mask=valid)
        plsc.addupdate_scatter(cursor, [qv], ones, mask=valid)
        return carry

    lax.fori_loop(0, CAP // LANES, place_chunk, 0)
    return boff


def _extract_window(slot3, par, wl2_j, wstart, e_lo, e_hi, outbuf, dslab):
    iot = lax.iota(jnp.int32, LANES)
    hi_half = (iot >= 8).astype(jnp.int32)
    lo_half = iot < 8
    pvec = jnp.full((LANES,), par, jnp.int32)

    def pair(p, carry):
        e = e_lo + 2 * p
        mask = lo_half | jnp.broadcast_to((e + 1) < e_hi, (LANES,))
        evec = e + hi_half
        jv = plsc.load_gather(wl2_j, [evec], mask=mask)
        tl = (jv >> 7) - wstart
        r = jv & 127
        jj = tl * 128 + r
        vals = plsc.load_gather(slot3, [pvec, iot & 7, jj], mask=mask)
        opos = evec * D + dslab * 8 + (iot & 7)
        plsc.store_scatter(outbuf, [opos], vals, mask=mask)
        return carry

    npairs = (e_hi - e_lo + 1) // 2
    lax.fori_loop(0, npairs, pair, 0)


def _stream_table(src, out_rows, wl_j, wl_i, wl2_j, wl2_i, stage, counts,
                  cursor, slot3, outbuf, sems, t0, t1, idx_hbm, wid):
    cnt = _filter_pass(idx_hbm, stage, wl_j, wl_i, t0, t1)
    boff = _bucket_sort(wl_j, wl_i, wl2_j, wl2_i, counts, cursor, cnt, t0, wid)
    bvec = boff
    cvec = counts[...]

    for b in range(NB):
        e_lo = bvec[b]
        e_hi = e_lo + cvec[b]
        wstart = jnp.minimum(t0 + b * W, NTILE - W)
        col = pl.multiple_of(wstart * 128, 128)

        def fire(dslab, par):
            d8 = pl.multiple_of(dslab * 8, 8)
            pltpu.async_copy(src.at[pl.ds(d8, 8), pl.ds(col, W * 128)],
                             slot3.at[par], sems[par])

        def drain(par):
            pltpu.make_async_copy(src.at[pl.ds(0, 8), pl.ds(0, W * 128)],
                                  slot3.at[par], sems[par]).wait()

        @pl.when(e_hi > e_lo)
        def _():
            fire(0, 0)

            def mbody(m, carry):
                fire(2 * m + 1, 1)
                drain(0)
                _extract_window(slot3, 0, wl2_j, wstart, e_lo, e_hi, outbuf, 2 * m)

                @pl.when(m < 3)
                def _():
                    fire(2 * m + 2, 0)
                drain(1)
                _extract_window(slot3, 1, wl2_j, wstart, e_lo, e_hi, outbuf, 2 * m + 1)
                return carry

            lax.fori_loop(0, 4, mbody, 0)

    pltpu.sync_copy(outbuf, out_rows.at[pl.ds(wid * (CAP * D), CAP * D)])


def _gather_body(u_hbm, v_hbm, ut_hbm, vt_hbm,
                 rows_u, rows_v, wli_u, wli_v,
                 stage, wl_j, wl_i, wl2_j, wl2_i, counts, cursor,
                 slot3, outbuf, sem_a, sem_b):
    wid = lax.axis_index("s") * NC + lax.axis_index("c")
    t0 = wid * TPW + jnp.minimum(wid, 5)
    t1 = t0 + TPW + jnp.where(wid < 5, 1, 0)
    sems = (sem_a, sem_b)
    _stream_table(ut_hbm, rows_u, wl_j, wl_i, wl2_j, wl2_i, stage, counts,
                  cursor, slot3, outbuf, sems, t0, t1, u_hbm, wid)
    pltpu.sync_copy(wl2_i, wli_u.at[pl.ds(wid * CAP, CAP)])
    _stream_table(vt_hbm, rows_v, wl_j, wl_i, wl2_j, wl2_i, stage, counts,
                  cursor, slot3, outbuf, sems, t0, t1, v_hbm, wid)
    pltpu.sync_copy(wl2_i, wli_v.at[pl.ds(wid * CAP, CAP)])


def _make_gather():
    mesh = plsc.VectorSubcoreMesh(core_axis_name="c", subcore_axis_name="s")
    return functools.partial(
        pl.kernel, mesh=mesh,
        compiler_params=pltpu.CompilerParams(use_tc_tiling_on_sc=True,
                                             needs_layout_passes=False),
        out_type=[jax.ShapeDtypeStruct((NW * CAP * D,), jnp.float32),
                  jax.ShapeDtypeStruct((NW * CAP * D,), jnp.float32),
                  jax.ShapeDtypeStruct((NW * CAP,), jnp.int32),
                  jax.ShapeDtypeStruct((NW * CAP,), jnp.int32)],
        scratch_types=[
            pltpu.VMEM((B // 2,), jnp.int32),
            pltpu.VMEM((CAP,), jnp.int32),
            pltpu.VMEM((CAP,), jnp.int32),
            pltpu.VMEM((CAP,), jnp.int32),
            pltpu.VMEM((CAP,), jnp.int32),
            pltpu.VMEM((LANES,), jnp.int32),
            pltpu.VMEM((LANES,), jnp.int32),
            pltpu.VMEM((2, 8, W * 128), jnp.float32),
            pltpu.VMEM((CAP * D,), jnp.float32),
            pltpu.SemaphoreType.DMA,
            pltpu.SemaphoreType.DMA,
        ],
    )(_gather_body)


_sc_gather = _make_gather()


def _scatter_body(rows_u, rows_v, wli_u, wli_v, ug_hbm, vg_hbm,
                  rowbuf, idxbuf, sem):
    wid = lax.axis_index("s") * NC + lax.axis_index("c")
    for rows, wli, out in ((rows_u, wli_u, ug_hbm), (rows_v, wli_v, vg_hbm)):
        pltpu.sync_copy(rows.at[pl.ds(wid * CAP, CAP)], rowbuf)
        pltpu.sync_copy(wli.at[wid], idxbuf)
        copies = []
        for k in range(CAP // 128):
            copies.append(pltpu.async_copy(
                rowbuf.at[pl.ds(k * 128, 128)], out.at[idxbuf.at[k]], sem))
        for c in copies:
            c.wait()


def _make_scatter():
    mesh = plsc.VectorSubcoreMesh(core_axis_name="c", subcore_axis_name="s")
    return functools.partial(
        pl.kernel, mesh=mesh,
        compiler_params=pltpu.CompilerParams(use_tc_tiling_on_sc=False),
        out_type=[jax.ShapeDtypeStruct((B + 128, D), jnp.float32),
                  jax.ShapeDtypeStruct((B + 128, D), jnp.float32)],
        scratch_types=[
            pltpu.VMEM((CAP, D), jnp.float32),
            pltpu.VMEM((CAP // 128, 128), jnp.int32),
            pltpu.SemaphoreType.DMA,
        ],
    )(_scatter_body)


_sc_scatter = _make_scatter()


def _mlp_body(xu_ref, xv_ref, w1a_ref, w1b_ref, b1_ref, w2_ref, b2_ref, o_ref):
    xu = jnp.maximum(xu_ref[...], 0.0)
    xv = jnp.maximum(xv_ref[...], 0.0)
    h = jnp.dot(xu, w1a_ref[...], preferred_element_type=jnp.float32)
    h = h + jnp.dot(xv, w1b_ref[...], preferred_element_type=jnp.float32)
    h = jnp.maximum(h + b1_ref[...], 0.0)
    o_ref[...] = jnp.dot(h, w2_ref[...], preferred_element_type=jnp.float32) + b2_ref[...]


BB = 2048


def _mlp(ug, vg, w1a, w1b, b1, w2, b2):
    grid = (B // BB,)
    return pl.pallas_call(
        _mlp_body,
        grid=grid,
        in_specs=[
            pl.BlockSpec((BB, D), lambda i: (i, 0)),
            pl.BlockSpec((BB, D), lambda i: (i, 0)),
            pl.BlockSpec((D, HIDDEN), lambda i: (0, 0)),
            pl.BlockSpec((D, HIDDEN), lambda i: (0, 0)),
            pl.BlockSpec((1, HIDDEN), lambda i: (0, 0)),
            pl.BlockSpec((HIDDEN, 1), lambda i: (0, 0)),
            pl.BlockSpec((1, 1), lambda i: (0, 0)),
        ],
        out_specs=pl.BlockSpec((BB, 1), lambda i: (i, 0)),
        out_shape=jax.ShapeDtypeStruct((B, 1), jnp.float32),
    )(ug, vg, w1a, w1b, b1, w2, b2)


def kernel(u, v, user_emb, like_emb, W1, b1, W2, b2):
    rows_u, rows_v, wli_u, wli_v = _sc_gather(u, v, user_emb.T, like_emb.T)
    ug, vg = _sc_scatter(rows_u.reshape(NW * CAP, D), rows_v.reshape(NW * CAP, D),
                         wli_u.reshape(NW, CAP // 128, 128),
                         wli_v.reshape(NW, CAP // 128, 128))
    return _mlp(ug, vg, W1[:D], W1[D:], b1.reshape(1, HIDDEN),
                W2, b2.reshape(1, 1))


# speedup vs baseline: 1.6557x; 1.0020x over previous
import functools

import jax
import jax.numpy as jnp
from jax import lax
from jax.experimental import pallas as pl
from jax.experimental.pallas import tpu as pltpu
from jax.experimental.pallas import tpu_sc as plsc

B = 16384
D = 64
HIDDEN = 256
NC = 2
NS = 16
NW = NC * NS
NTILE = 7813
TPW = 244
W = 24
NB = 11
CAP = 768
LANES = 16


def _scan_last(vec):
    return plsc.cumsum(vec)[LANES - 1]


def _filter_pass(src_hbm, stage, wl_j, wl_i, t0, t1):
    cnt = 0
    for half in range(2):
        pltpu.sync_copy(src_hbm.at[pl.ds(half * (B // 2), B // 2)], stage)

        def chunk(c, cnt):
            j = stage[pl.ds(pl.multiple_of(c * LANES, 8), LANES)]
            t = j >> 7
            mask = (t >= t0) & (t < t1)
            mi = mask.astype(jnp.int32)
            rank = plsc.cumsum(mi) - mi
            pos = rank + cnt
            mask = mask & (pos < CAP)
            ivec = lax.iota(jnp.int32, LANES) + (half * (B // 2) + c * LANES)
            plsc.store_scatter(wl_j, [pos], j, mask=mask)
            plsc.store_scatter(wl_i, [pos], ivec, mask=mask)
            return cnt + _scan_last(mi)

        cnt = lax.fori_loop(0, (B // 2) // LANES, chunk, cnt)
    return cnt


def _bucket_sort(wl_j, wl_i, wl2_j, wl2_i, counts, cursor, cnt, t0, wid):
    zeros = jnp.zeros((LANES,), jnp.int32)
    counts[...] = zeros
    pad = jnp.full((LANES,), B, jnp.int32) + wid
    for c in range(CAP // LANES):
        wl2_i[pl.ds(c * LANES, LANES)] = pad
    ones = jnp.full((LANES,), 1, jnp.int32)

    def count_chunk(c, carry):
        off = pl.multiple_of(c * LANES, 8)
        j = wl_j[pl.ds(off, LANES)]
        valid = (lax.iota(jnp.int32, LANES) + c * LANES) < cnt
        qv = ((j >> 7) - t0) // W
        plsc.addupdate_scatter(counts, [qv], ones, mask=valid)
        return carry

    lax.fori_loop(0, CAP // LANES, count_chunk, 0)
    cvec = counts[...]
    incl = plsc.cumsum(cvec)
    boff = incl - cvec
    cursor[...] = boff

    def place_chunk(c, carry):
        off = pl.multiple_of(c * LANES, 8)
        j = wl_j[pl.ds(off, LANES)]
        i = wl_i[pl.ds(off, LANES)]
        valid = (lax.iota(jnp.int32, LANES) + c * LANES) < cnt
        qv = ((j >> 7) - t0) // W
        base = plsc.load_gather(cursor, [qv], mask=valid)
        rank = zeros
        for q in range(NB):
            mq = (valid & (qv == q)).astype(jnp.int32)
            rank = rank + (plsc.cumsum(mq) - mq) * mq
        pos = base + rank
        plsc.store_scatter(wl2_j, [pos], j, mask=valid)
        plsc.store_scatter(wl2_i, [pos], i, mask=valid)
        plsc.addupdate_scatter(cursor, [qv], ones, mask=valid)
        return carry

    lax.fori_loop(0, CAP // LANES, place_chunk, 0)
    return boff


def _extract_window(slot3, par, wl2_j, wstart, e_lo, e_hi, outbuf, dslab):
    iot = lax.iota(jnp.int32, LANES)
    hi_half = (iot >= 8).astype(jnp.int32)
    lo_half = iot < 8
    pvec = jnp.full((LANES,), par, jnp.int32)

    def quad(p, carry):
        e = e_lo + 4 * p
        for h in range(2):
            eh = e + 2 * h
            mask = lo_half | jnp.broadcast_to((eh + 1) < e_hi, (LANES,))
            if h:
                mask = mask & jnp.broadcast_to(eh < e_hi, (LANES,))
            evec = eh + hi_half
            jv = plsc.load_gather(wl2_j, [evec], mask=mask)
            tl = (jv >> 7) - wstart
            r = jv & 127
            jj = tl * 128 + r
            vals = plsc.load_gather(slot3, [pvec, iot & 7, jj], mask=mask)
            opos = evec * D + dslab * 8 + (iot & 7)
            plsc.store_scatter(outbuf, [opos], vals, mask=mask)
        return carry

    nquads = (e_hi - e_lo + 3) // 4
    lax.fori_loop(0, nquads, quad, 0)


def _stream_table(src, out_rows, wl_j, wl_i, wl2_j, wl2_i, stage, counts,
                  cursor, slot3, outbuf, sems, t0, t1, idx_hbm, wid):
    cnt = _filter_pass(idx_hbm, stage, wl_j, wl_i, t0, t1)
    boff = _bucket_sort(wl_j, wl_i, wl2_j, wl2_i, counts, cursor, cnt, t0, wid)
    bvec = boff
    cvec = counts[...]

    for b in range(NB):
        e_lo = bvec[b]
        e_hi = e_lo + cvec[b]
        wstart = jnp.minimum(t0 + b * W, NTILE - W)
        col = pl.multiple_of(wstart * 128, 128)

        def fire(dslab, par):
            d8 = pl.multiple_of(dslab * 8, 8)
            pltpu.async_copy(src.at[pl.ds(d8, 8), pl.ds(col, W * 128)],
                             slot3.at[par], sems[par])

        def drain(par):
            pltpu.make_async_copy(src.at[pl.ds(0, 8), pl.ds(0, W * 128)],
                                  slot3.at[par], sems[par]).wait()

        @pl.when(e_hi > e_lo)
        def _():
            fire(0, 0)

            def mbody(m, carry):
                fire(2 * m + 1, 1)
                drain(0)
                _extract_window(slot3, 0, wl2_j, wstart, e_lo, e_hi, outbuf, 2 * m)

                @pl.when(m < 3)
                def _():
                    fire(2 * m + 2, 0)
                drain(1)
                _extract_window(slot3, 1, wl2_j, wstart, e_lo, e_hi, outbuf, 2 * m + 1)
                return carry

            lax.fori_loop(0, 4, mbody, 0)

    pltpu.sync_copy(outbuf, out_rows.at[pl.ds(wid * (CAP * D), CAP * D)])


def _gather_body(u_hbm, v_hbm, ut_hbm, vt_hbm,
                 rows_u, rows_v, wli_u, wli_v,
                 stage, wl_j, wl_i, wl2_j, wl2_i, counts, cursor,
                 slot3, outbuf, sem_a, sem_b):
    wid = lax.axis_index("s") * NC + lax.axis_index("c")
    t0 = wid * TPW + jnp.minimum(wid, 5)
    t1 = t0 + TPW + jnp.where(wid < 5, 1, 0)
    sems = (sem_a, sem_b)
    _stream_table(ut_hbm, rows_u, wl_j, wl_i, wl2_j, wl2_i, stage, counts,
                  cursor, slot3, outbuf, sems, t0, t1, u_hbm, wid)
    pltpu.sync_copy(wl2_i, wli_u.at[pl.ds(wid * CAP, CAP)])
    _stream_table(vt_hbm, rows_v, wl_j, wl_i, wl2_j, wl2_i, stage, counts,
                  cursor, slot3, outbuf, sems, t0, t1, v_hbm, wid)
    pltpu.sync_copy(wl2_i, wli_v.at[pl.ds(wid * CAP, CAP)])


def _make_gather():
    mesh = plsc.VectorSubcoreMesh(core_axis_name="c", subcore_axis_name="s")
    return functools.partial(
        pl.kernel, mesh=mesh,
        compiler_params=pltpu.CompilerParams(use_tc_tiling_on_sc=True,
                                             needs_layout_passes=False),
        out_type=[jax.ShapeDtypeStruct((NW * CAP * D,), jnp.float32),
                  jax.ShapeDtypeStruct((NW * CAP * D,), jnp.float32),
                  jax.ShapeDtypeStruct((NW * CAP,), jnp.int32),
                  jax.ShapeDtypeStruct((NW * CAP,), jnp.int32)],
        scratch_types=[
            pltpu.VMEM((B // 2,), jnp.int32),
            pltpu.VMEM((CAP,), jnp.int32),
            pltpu.VMEM((CAP,), jnp.int32),
            pltpu.VMEM((CAP,), jnp.int32),
            pltpu.VMEM((CAP,), jnp.int32),
            pltpu.VMEM((LANES,), jnp.int32),
            pltpu.VMEM((LANES,), jnp.int32),
            pltpu.VMEM((2, 8, W * 128), jnp.float32),
            pltpu.VMEM((CAP * D,), jnp.float32),
            pltpu.SemaphoreType.DMA,
            pltpu.SemaphoreType.DMA,
        ],
    )(_gather_body)


_sc_gather = _make_gather()


def _scatter_body(rows_u, rows_v, wli_u, wli_v, ug_hbm, vg_hbm,
                  rowbuf, idxbuf, sem):
    wid = lax.axis_index("s") * NC + lax.axis_index("c")
    for rows, wli, out in ((rows_u, wli_u, ug_hbm), (rows_v, wli_v, vg_hbm)):
        pltpu.sync_copy(rows.at[pl.ds(wid * CAP, CAP)], rowbuf)
        pltpu.sync_copy(wli.at[wid], idxbuf)
        copies = []
        for k in range(CAP // 128):
            copies.append(pltpu.async_copy(
                rowbuf.at[pl.ds(k * 128, 128)], out.at[idxbuf.at[k]], sem))
        for c in copies:
            c.wait()


def _make_scatter():
    mesh = plsc.VectorSubcoreMesh(core_axis_name="c", subcore_axis_name="s")
    return functools.partial(
        pl.kernel, mesh=mesh,
        compiler_params=pltpu.CompilerParams(use_tc_tiling_on_sc=False),
        out_type=[jax.ShapeDtypeStruct((B + 128, D), jnp.float32),
                  jax.ShapeDtypeStruct((B + 128, D), jnp.float32)],
        scratch_types=[
            pltpu.VMEM((CAP, D), jnp.float32),
            pltpu.VMEM((CAP // 128, 128), jnp.int32),
            pltpu.SemaphoreType.DMA,
        ],
    )(_scatter_body)


_sc_scatter = _make_scatter()


def _mlp_body(xu_ref, xv_ref, w1a_ref, w1b_ref, b1_ref, w2_ref, b2_ref, o_ref):
    xu = jnp.maximum(xu_ref[...], 0.0)
    xv = jnp.maximum(xv_ref[...], 0.0)
    h = jnp.dot(xu, w1a_ref[...], preferred_element_type=jnp.float32)
    h = h + jnp.dot(xv, w1b_ref[...], preferred_element_type=jnp.float32)
    h = jnp.maximum(h + b1_ref[...], 0.0)
    o_ref[...] = jnp.dot(h, w2_ref[...], preferred_element_type=jnp.float32) + b2_ref[...]


BB = 2048


def _mlp(ug, vg, w1a, w1b, b1, w2, b2):
    grid = (B // BB,)
    return pl.pallas_call(
        _mlp_body,
        grid=grid,
        in_specs=[
            pl.BlockSpec((BB, D), lambda i: (i, 0)),
            pl.BlockSpec((BB, D), lambda i: (i, 0)),
            pl.BlockSpec((D, HIDDEN), lambda i: (0, 0)),
            pl.BlockSpec((D, HIDDEN), lambda i: (0, 0)),
            pl.BlockSpec((1, HIDDEN), lambda i: (0, 0)),
            pl.BlockSpec((HIDDEN, 1), lambda i: (0, 0)),
            pl.BlockSpec((1, 1), lambda i: (0, 0)),
        ],
        out_specs=pl.BlockSpec((BB, 1), lambda i: (i, 0)),
        out_shape=jax.ShapeDtypeStruct((B, 1), jnp.float32),
    )(ug, vg, w1a, w1b, b1, w2, b2)


def kernel(u, v, user_emb, like_emb, W1, b1, W2, b2):
    rows_u, rows_v, wli_u, wli_v = _sc_gather(u, v, user_emb.T, like_emb.T)
    ug, vg = _sc_scatter(rows_u.reshape(NW * CAP, D), rows_v.reshape(NW * CAP, D),
                         wli_u.reshape(NW, CAP // 128, 128),
                         wli_v.reshape(NW, CAP // 128, 128))
    return _mlp(ug, vg, W1[:D], W1[D:], b1.reshape(1, HIDDEN),
                W2, b2.reshape(1, 1))


# speedup vs baseline: 1.9985x; 1.2071x over previous
import functools

import jax
import jax.numpy as jnp
from jax import lax
from jax.experimental import pallas as pl
from jax.experimental.pallas import tpu as pltpu
from jax.experimental.pallas import tpu_sc as plsc

B = 16384
D = 64
HIDDEN = 256
NC = 2
NS = 16
NW = NC * NS
NTILE = 7813
TPW = 244
W = 8
NB = 32
CAP = 768
LANES = 16


def _scan_last(vec):
    return plsc.cumsum(vec)[LANES - 1]


def _filter_pass(src_hbm, stage, wl_j, wl_i, t0, t1):
    cnt = 0
    for half in range(4):
        pltpu.sync_copy(src_hbm.at[pl.ds(half * (B // 4), B // 4)], stage)

        def chunk(c, cnt):
            j = stage[pl.ds(pl.multiple_of(c * LANES, 8), LANES)]
            t = j >> 7
            mask = (t >= t0) & (t < t1)
            mi = mask.astype(jnp.int32)
            s = plsc.cumsum(mi)
            pos = (s - mi) + cnt
            mask = mask & (pos < CAP)
            ivec = lax.iota(jnp.int32, LANES) + (half * (B // 4) + c * LANES)
            plsc.store_scatter(wl_j, [pos], j, mask=mask)
            plsc.store_scatter(wl_i, [pos], ivec, mask=mask)
            return cnt + s[LANES - 1]

        cnt = lax.fori_loop(0, (B // 4) // LANES, chunk, cnt)
    return cnt


def _bucket_sort(wl_j, wl_i, wl2_j, wl2_i, counts, cursor, boffv, cnt, t0, wid):
    zeros = jnp.zeros((LANES,), jnp.int32)
    counts[pl.ds(0, LANES)] = zeros
    counts[pl.ds(LANES, LANES)] = zeros
    pad = jnp.full((LANES,), B, jnp.int32) + wid
    for c in range(CAP // LANES):
        wl2_i[pl.ds(c * LANES, LANES)] = pad
    ones = jnp.full((LANES,), 1, jnp.int32)

    def count_chunk(c, carry):
        off = pl.multiple_of(c * LANES, 8)
        j = wl_j[pl.ds(off, LANES)]
        valid = (lax.iota(jnp.int32, LANES) + c * LANES) < cnt
        qv = ((j >> 7) - t0) // W
        plsc.addupdate_scatter(counts, [qv], ones, mask=valid)
        return carry

    lax.fori_loop(0, CAP // LANES, count_chunk, 0)
    c0 = counts[pl.ds(0, LANES)]
    c1 = counts[pl.ds(LANES, LANES)]
    i0 = plsc.cumsum(c0)
    b0 = i0 - c0
    b1 = (plsc.cumsum(c1) - c1) + i0[LANES - 1]
    cursor[pl.ds(0, LANES)] = b0
    cursor[pl.ds(LANES, LANES)] = b1
    boffv[pl.ds(0, LANES)] = b0
    boffv[pl.ds(LANES, LANES)] = b1

    def place_chunk(c, carry):
        off = pl.multiple_of(c * LANES, 8)
        j = wl_j[pl.ds(off, LANES)]
        i = wl_i[pl.ds(off, LANES)]
        valid = (lax.iota(jnp.int32, LANES) + c * LANES) < cnt
        qv = ((j >> 7) - t0) // W
        base = plsc.load_gather(cursor, [qv], mask=valid)
        rank = zeros
        for q in range(NB):
            mq = (valid & (qv == q)).astype(jnp.int32)
            rank = rank + (plsc.cumsum(mq) - mq) * mq
        pos = base + rank
        plsc.store_scatter(wl2_j, [pos], j, mask=valid)
        plsc.store_scatter(wl2_i, [pos], i, mask=valid)
        plsc.addupdate_scatter(cursor, [qv], ones, mask=valid)
        return carry

    lax.fori_loop(0, CAP // LANES, place_chunk, 0)


def _extract_window(slot3, par, wl2_j, wstart, e_lo, e_hi, outbuf, dslab):
    iot = lax.iota(jnp.int32, LANES)
    hi_half = (iot >= 8).astype(jnp.int32)
    lo_half = iot < 8
    pvec = jnp.full((LANES,), par, jnp.int32)

    def quad(p, carry):
        e = e_lo + 4 * p
        for h in range(2):
            eh = e + 2 * h
            mask = lo_half | jnp.broadcast_to((eh + 1) < e_hi, (LANES,))
            if h:
                mask = mask & jnp.broadcast_to(eh < e_hi, (LANES,))
            evec = eh + hi_half
            jv = plsc.load_gather(wl2_j, [evec], mask=mask)
            tl = (jv >> 7) - wstart
            r = jv & 127
            jj = tl * 128 + r
            vals = plsc.load_gather(slot3, [pvec, iot & 7, jj], mask=mask)
            opos = evec * D + dslab * 8 + (iot & 7)
            plsc.store_scatter(outbuf, [opos], vals, mask=mask)
        return carry

    nquads = (e_hi - e_lo + 3) // 4
    lax.fori_loop(0, nquads, quad, 0)


def _stream_table(src, out_rows, wl_j, wl_i, wl2_j, wl2_i, stage, counts,
                  cursor, boffv, slots, outbuf, sems, t0, t1, idx_hbm, wid):
    cnt = _filter_pass(idx_hbm, stage, wl_j, wl_i, t0, t1)
    _bucket_sort(wl_j, wl_i, wl2_j, wl2_i, counts, cursor, boffv, cnt, t0, wid)

    def fire_at(col, s):
        pltpu.async_copy(src.at[pl.ds(s * 8, 8), pl.ds(col, W * 128)],
                         slots.at[s], sems[s])

    def drain(s):
        pltpu.make_async_copy(src.at[pl.ds(0, 8), pl.ds(0, W * 128)],
                              slots.at[s], sems[s]).wait()

    col0 = pl.multiple_of(t0 * 128, 128)
    for s in range(8):
        fire_at(col0, s)

    def bbody(b, carry):
        bsp = jnp.full((LANES,), b, jnp.int32)
        e_lo = plsc.load_gather(boffv, [bsp])[0]
        e_hi = e_lo + plsc.load_gather(counts, [bsp])[0]
        wstart = jnp.minimum(t0 + b * W, NTILE - W)
        ncol = pl.multiple_of(
            jnp.minimum(t0 + (b + 1) * W, NTILE - W) * 128, 128)
        for s in range(8):
            drain(s)
            _extract_window(slots, s, wl2_j, wstart, e_lo, e_hi, outbuf, s)

            @pl.when(b + 1 < NB)
            def _(ss=s, nc=ncol):
                fire_at(nc, ss)
        return carry

    lax.fori_loop(0, NB, bbody, 0)
    pltpu.sync_copy(outbuf, out_rows.at[pl.ds(wid * (CAP * D), CAP * D)])


def _gather_body(u_hbm, v_hbm, ut_hbm, vt_hbm,
                 rows_u, rows_v, wli_u, wli_v,
                 stage, wl_j, wl_i, wl2_j, wl2_i, counts, cursor, boffv,
                 slots, outbuf, *sems):
    wid = lax.axis_index("s") * NC + lax.axis_index("c")
    t0 = wid * TPW + jnp.minimum(wid, 5)
    t1 = t0 + TPW + jnp.where(wid < 5, 1, 0)
    _stream_table(ut_hbm, rows_u, wl_j, wl_i, wl2_j, wl2_i, stage, counts,
                  cursor, boffv, slots, outbuf, sems, t0, t1, u_hbm, wid)
    pltpu.sync_copy(wl2_i, wli_u.at[pl.ds(wid * CAP, CAP)])
    _stream_table(vt_hbm, rows_v, wl_j, wl_i, wl2_j, wl2_i, stage, counts,
                  cursor, boffv, slots, outbuf, sems, t0, t1, v_hbm, wid)
    pltpu.sync_copy(wl2_i, wli_v.at[pl.ds(wid * CAP, CAP)])


def _make_gather():
    mesh = plsc.VectorSubcoreMesh(core_axis_name="c", subcore_axis_name="s")
    return functools.partial(
        pl.kernel, mesh=mesh,
        compiler_params=pltpu.CompilerParams(use_tc_tiling_on_sc=True,
                                             needs_layout_passes=False),
        out_type=[jax.ShapeDtypeStruct((NW * CAP * D,), jnp.float32),
                  jax.ShapeDtypeStruct((NW * CAP * D,), jnp.float32),
                  jax.ShapeDtypeStruct((NW * CAP,), jnp.int32),
                  jax.ShapeDtypeStruct((NW * CAP,), jnp.int32)],
        scratch_types=[
            pltpu.VMEM((B // 4,), jnp.int32),
            pltpu.VMEM((CAP,), jnp.int32),
            pltpu.VMEM((CAP,), jnp.int32),
            pltpu.VMEM((CAP,), jnp.int32),
            pltpu.VMEM((CAP,), jnp.int32),
            pltpu.VMEM((NB,), jnp.int32),
            pltpu.VMEM((NB,), jnp.int32),
            pltpu.VMEM((NB,), jnp.int32),
            pltpu.VMEM((8, 8, W * 128), jnp.float32),
            pltpu.VMEM((CAP * D,), jnp.float32),
        ] + [pltpu.SemaphoreType.DMA] * 8,
    )(_gather_body)


_sc_gather = _make_gather()


def _scatter_body(rows_u, rows_v, wli_u, wli_v, ug_hbm, vg_hbm,
                  rowbuf, idxbuf, sem):
    wid = lax.axis_index("s") * NC + lax.axis_index("c")
    for rows, wli, out in ((rows_u, wli_u, ug_hbm), (rows_v, wli_v, vg_hbm)):
        pltpu.sync_copy(rows.at[pl.ds(wid * CAP, CAP)], rowbuf)
        pltpu.sync_copy(wli.at[wid], idxbuf)
        copies = []
        for k in range(CAP // 128):
            copies.append(pltpu.async_copy(
                rowbuf.at[pl.ds(k * 128, 128)], out.at[idxbuf.at[k]], sem))
        for c in copies:
            c.wait()


def _make_scatter():
    mesh = plsc.VectorSubcoreMesh(core_axis_name="c", subcore_axis_name="s")
    return functools.partial(
        pl.kernel, mesh=mesh,
        compiler_params=pltpu.CompilerParams(use_tc_tiling_on_sc=False),
        out_type=[jax.ShapeDtypeStruct((B + 128, D), jnp.float32),
                  jax.ShapeDtypeStruct((B + 128, D), jnp.float32)],
        scratch_types=[
            pltpu.VMEM((CAP, D), jnp.float32),
            pltpu.VMEM((CAP // 128, 128), jnp.int32),
            pltpu.SemaphoreType.DMA,
        ],
    )(_scatter_body)


_sc_scatter = _make_scatter()


def _mlp_body(xu_ref, xv_ref, w1a_ref, w1b_ref, b1_ref, w2_ref, b2_ref, o_ref):
    xu = jnp.maximum(xu_ref[...], 0.0)
    xv = jnp.maximum(xv_ref[...], 0.0)
    h = jnp.dot(xu, w1a_ref[...], preferred_element_type=jnp.float32)
    h = h + jnp.dot(xv, w1b_ref[...], preferred_element_type=jnp.float32)
    h = jnp.maximum(h + b1_ref[...], 0.0)
    o_ref[...] = jnp.dot(h, w2_ref[...], preferred_element_type=jnp.float32) + b2_ref[...]


BB = 2048


def _mlp(ug, vg, w1a, w1b, b1, w2, b2):
    grid = (B // BB,)
    return pl.pallas_call(
        _mlp_body,
        grid=grid,
        in_specs=[
            pl.BlockSpec((BB, D), lambda i: (i, 0)),
            pl.BlockSpec((BB, D), lambda i: (i, 0)),
            pl.BlockSpec((D, HIDDEN), lambda i: (0, 0)),
            pl.BlockSpec((D, HIDDEN), lambda i: (0, 0)),
            pl.BlockSpec((1, HIDDEN), lambda i: (0, 0)),
            pl.BlockSpec((HIDDEN, 1), lambda i: (0, 0)),
            pl.BlockSpec((1, 1), lambda i: (0, 0)),
        ],
        out_specs=pl.BlockSpec((BB, 1), lambda i: (i, 0)),
        out_shape=jax.ShapeDtypeStruct((B, 1), jnp.float32),
    )(ug, vg, w1a, w1b, b1, w2, b2)


def kernel(u, v, user_emb, like_emb, W1, b1, W2, b2):
    rows_u, rows_v, wli_u, wli_v = _sc_gather(u, v, user_emb.T, like_emb.T)
    ug, vg = _sc_scatter(rows_u.reshape(NW * CAP, D), rows_v.reshape(NW * CAP, D),
                         wli_u.reshape(NW, CAP // 128, 128),
                         wli_v.reshape(NW, CAP // 128, 128))
    return _mlp(ug, vg, W1[:D], W1[D:], b1.reshape(1, HIDDEN),
                W2, b2.reshape(1, 1))
